# baseline (speedup 1.0000x reference)
import functools

import jax
import jax.numpy as jnp
import numpy as np
from jax.experimental import pallas as pl

N0 = 50000
N1 = 10242
NP1 = 10496
CIN = 128
CC = 16
HID = 128
LAT = 32
HEADS = 8
DH = HID // HEADS
INV_SQRT_DH = 1.0 / float(np.sqrt(DH))
SAT_BOUND = 5.0

_MH = np.zeros((HID, HEADS), np.float32)
for _d in range(HID):
    _MH[_d, _d // DH] = 1.0
MH = jnp.asarray(_MH)
MHT = jnp.asarray(_MH.T.copy())


def _layernorm(x):
    m = x.mean(-1, keepdims=True)
    v = x.var(-1, keepdims=True)
    return (x - m) * jax.lax.rsqrt(v + 1e-5)


def _dot(a, b):
    return jnp.dot(a, b, preferred_element_type=jnp.float32)


def _attn(q, kk_ref, vv_ref, mh, mht, nk):
    den = None
    acc = None
    for k in range(nk):
        kkk = kk_ref[:, k, :]
        e = jnp.exp(_dot(q * kkk, mh))
        den = e if den is None else den + e
        contrib = _dot(e, mht) * vv_ref[:, k, :]
        acc = contrib if acc is None else acc + contrib
    return acc * _dot(1.0 / den, mht)


def _full(shape):
    return pl.BlockSpec(shape, lambda i: tuple(0 for _ in shape))


def _rows(block, width):
    return pl.BlockSpec((block, width), lambda i: (i, 0))


def _rows3(block, k, width):
    return pl.BlockSpec((block, k, width), lambda i: (i, 0, 0))


def _f32(shape):
    return jax.ShapeDtypeStruct(shape, jnp.float32)


def _stage_a_body(x_ref, feat_ref, wce_ref, bce_ref, wq_ref, wk_ref, wv_ref,
                  ws_ref, q_ref, k_ref, v_ref, s_ref):
    x0 = x_ref[...] + jnp.sin(_dot(feat_ref[...], wce_ref[...]) + bce_ref[...])
    ln = _layernorm(x0)
    q_ref[...] = _dot(ln, wq_ref[...]) * INV_SQRT_DH
    k_ref[...] = _dot(ln, wk_ref[...])
    v_ref[...] = _dot(ln, wv_ref[...])
    s_ref[...] = _dot(x0, ws_ref[...])


def _stage_a(x, feat, wce, bce, wq, wk, wv, ws, block=2000):
    n = x.shape[0]
    f = feat.shape[1]
    return pl.pallas_call(
        _stage_a_body,
        grid=(n // block,),
        in_specs=[_rows(block, CIN), _rows(block, f), _full((f, CIN)),
                  _full((1, CIN)), _full((CIN, HID)), _full((CIN, HID)),
                  _full((CIN, HID)), _full((CIN, HID))],
        out_specs=[_rows(block, HID)] * 4,
        out_shape=[_f32((n, HID))] * 4,
    )(x, feat, wce, bce, wq, wk, wv, ws)


def _stage_b2_body(q0_ref, s0_ref, feat_ref, kk_ref, vv_ref, mh_ref, mht_ref,
                   wo_ref, wce_ref, bce_ref, wq_ref, wk_ref, wv_ref,
                   h_ref, q_ref, k_ref, v_ref, *, nk):
    at = _attn(q0_ref[...], kk_ref, vv_ref, mh_ref[...], mht_ref[...], nk)
    h = s0_ref[...] + _dot(at, wo_ref[...]) \
        + jnp.sin(_dot(feat_ref[...], wce_ref[...]) + bce_ref[...])
    h_ref[...] = h
    hn = _layernorm(h)
    q_ref[...] = _dot(hn, wq_ref[...]) * INV_SQRT_DH
    k_ref[...] = _dot(hn, wk_ref[...])
    v_ref[...] = _dot(hn, wv_ref[...])


def _stage_b2(q0, s0, feat, kk, vv, wo, wce, bce, wq, wk, wv, block=328):
    n, nk, _ = kk.shape
    f = feat.shape[1]
    return pl.pallas_call(
        functools.partial(_stage_b2_body, nk=nk),
        grid=(n // block,),
        in_specs=[_rows(block, HID), _rows(block, HID), _rows(block, f),
                  _rows3(block, nk, HID), _rows3(block, nk, HID),
                  _full((HID, HEADS)), _full((HEADS, HID)),
                  _full((HID, HID)), _full((f, HID)), _full((1, HID)),
                  _full((HID, HID)), _full((HID, HID)), _full((HID, HID))],
        out_specs=[_rows(block, HID)] * 4,
        out_shape=[_f32((n, HID))] * 4,
    )(q0, s0, feat, kk, vv, MH, MHT, wo, wce, bce, wq, wk, wv)


def _stage_sa_body(h_ref, q_ref, kk_ref, vv_ref, mh_ref, mht_ref, wo_ref,
                   wq_ref, wk_ref, wv_ref,
                   h2_ref, q2_ref, k2_ref, v2_ref, *, nk):
    at = _attn(q_ref[...], kk_ref, vv_ref, mh_ref[...], mht_ref[...], nk)
    h2 = h_ref[...] + _dot(at, wo_ref[...])
    h2_ref[...] = h2
    hn = _layernorm(h2)
    q2_ref[...] = _dot(hn, wq_ref[...]) * INV_SQRT_DH
    k2_ref[...] = _dot(hn, wk_ref[...])
    v2_ref[...] = _dot(hn, wv_ref[...])


def _stage_sa(h, q, kk, vv, wo, wq, wk, wv, block=328):
    n, nk, _ = kk.shape
    return pl.pallas_call(
        functools.partial(_stage_sa_body, nk=nk),
        grid=(n // block,),
        in_specs=[_rows(block, HID), _rows(block, HID),
                  _rows3(block, nk, HID), _rows3(block, nk, HID),
                  _full((HID, HEADS)), _full((HEADS, HID)),
                  _full((HID, HID)), _full((HID, HID)), _full((HID, HID)),
                  _full((HID, HID))],
        out_specs=[_rows(block, HID)] * 4,
        out_shape=[_f32((n, HID))] * 4,
    )(h, q, kk, vv, MH, MHT, wo, wq, wk, wv)


def _stage_b4_body(h_ref, q_ref, kk_ref, vv_ref, mh_ref, mht_ref, wo_ref,
                   wlat_ref, blat_ref, wdlat_ref, bdlat_ref, feat_ref,
                   wcd_ref, bcd_ref, wq_ref, wk_ref, wv_ref,
                   g_ref, q2_ref, k2_ref, v2_ref, *, nk):
    at = _attn(q_ref[...], kk_ref, vv_ref, mh_ref[...], mht_ref[...], nk)
    h3 = h_ref[...] + _dot(at, wo_ref[...])
    z = _dot(h3, wlat_ref[...]) + blat_ref[...]
    z = z * jax.lax.rsqrt(1.0 + (z / SAT_BOUND) ** 2)
    g = _dot(z, wdlat_ref[...]) + bdlat_ref[...]
    g = g + jnp.sin(_dot(feat_ref[...], wcd_ref[...]) + bcd_ref[...])
    g_ref[...] = g
    gn = _layernorm(g)
    q2_ref[...] = _dot(gn, wq_ref[...]) * INV_SQRT_DH
    k2_ref[...] = _dot(gn, wk_ref[...])
    v2_ref[...] = _dot(gn, wv_ref[...])


def _stage_b4(h, q, kk, vv, wo, wlat, blat, wdlat, bdlat, feat, wcd, bcd,
              wq, wk, wv, block=328):
    n, nk, _ = kk.shape
    f = feat.shape[1]
    return pl.pallas_call(
        functools.partial(_stage_b4_body, nk=nk),
        grid=(n // block,),
        in_specs=[_rows(block, HID), _rows(block, HID),
                  _rows3(block, nk, HID), _rows3(block, nk, HID),
                  _full((HID, HEADS)), _full((HEADS, HID)),
                  _full((HID, HID)),
                  _full((HID, LAT)), _full((1, LAT)),
                  _full((LAT, HID)), _full((1, HID)),
                  _rows(block, f), _full((f, HID)), _full((1, HID)),
                  _full((HID, HID)), _full((HID, HID)), _full((HID, HID))],
        out_specs=[_rows(block, HID)] * 4,
        out_shape=[_f32((n, HID))] * 4,
    )(h, q, kk, vv, MH, MHT, wo, wlat, blat, wdlat, bdlat, feat, wcd, bcd,
      wq, wk, wv)


def _stage_b6_body(h_ref, q_ref, kk_ref, vv_ref, mh_ref, mht_ref, wo_ref,
                   wq_ref, wk_ref, wv_ref, ws_ref,
                   q2_ref, k2_ref, v2_ref, s2_ref, *, nk):
    at = _attn(q_ref[...], kk_ref, vv_ref, mh_ref[...], mht_ref[...], nk)
    g3 = h_ref[...] + _dot(at, wo_ref[...])
    gn = _layernorm(g3)
    q2_ref[...] = _dot(gn, wq_ref[...]) * INV_SQRT_DH
    k2_ref[...] = _dot(gn, wk_ref[...])
    v2_ref[...] = _dot(gn, wv_ref[...])
    s2_ref[...] = _dot(g3, ws_ref[...])


def _stage_b6(h, q, kk, vv, wo, wq, wk, wv, ws, block=328):
    n, nk, _ = kk.shape
    return pl.pallas_call(
        functools.partial(_stage_b6_body, nk=nk),
        grid=(n // block,),
        in_specs=[_rows(block, HID), _rows(block, HID),
                  _rows3(block, nk, HID), _rows3(block, nk, HID),
                  _full((HID, HEADS)), _full((HEADS, HID)),
                  _full((HID, HID)), _full((HID, HID)), _full((HID, HID)),
                  _full((HID, HID)), _full((HID, HID))],
        out_specs=[_rows(block, HID)] * 4,
        out_shape=[_f32((n, HID))] * 4,
    )(h, q, kk, vv, MH, MHT, wo, wq, wk, wv, ws)


def _stage_c_body(s_ref, q_ref, kk_ref, vv_ref, mh_ref, mht_ref, wo_ref,
                  out_ref, *, nk):
    at = _attn(q_ref[...], kk_ref, vv_ref, mh_ref[...], mht_ref[...], nk)
    out_ref[...] = s_ref[...] + _dot(at, wo_ref[...])


def _stage_c(s, q, kk, vv, wo, block=1000):
    n, nk, _ = kk.shape
    return pl.pallas_call(
        functools.partial(_stage_c_body, nk=nk),
        grid=(n // block,),
        in_specs=[_rows(block, HID), _rows(block, HID),
                  _rows3(block, nk, HID), _rows3(block, nk, HID),
                  _full((HID, HEADS)), _full((HEADS, HID)),
                  _full((HID, HID))],
        out_specs=_rows(block, CIN),
        out_shape=_f32((n, CIN)),
    )(s, q, kk, vv, MH, MHT, wo)


def _take(table, idx):
    return jnp.take(table, idx, axis=0)


def kernel(x, context, coord4_grid, coord4_ico, params, pool_nbrs, sa_nbrs,
           dec_nbrs, enc_q_idx, dec_q_idx):
    pe = params["enc_pool"]
    pd = params["dec_pool"]

    feat_grid = jnp.concatenate([coord4_grid, context], axis=-1)
    Q0, K0, V0, S0 = _stage_a(x, feat_grid, params["Wce0"],
                              params["bce0"].reshape(1, CIN),
                              pe["Wq"], pe["Wk"], pe["Wv"], pe["Wskip"])

    pad1 = NP1 - N1
    enc_q_p = jnp.pad(enc_q_idx, (0, pad1))
    sa_p = jnp.pad(sa_nbrs, ((0, pad1), (0, 0)))
    pool_p = jnp.pad(pool_nbrs, ((0, pad1), (0, 0)))

    q0 = _take(Q0, enc_q_p)
    s0 = _take(S0, enc_q_p)
    kk0 = _take(K0, pool_p.reshape(-1)).reshape(NP1, -1, HID)
    vv0 = _take(V0, pool_p.reshape(-1)).reshape(NP1, -1, HID)

    ctx_ico = _take(context, enc_q_p)
    feat_ico = jnp.concatenate(
        [jnp.pad(coord4_ico, ((0, pad1), (0, 0))), ctx_ico], axis=-1)

    p1, p2 = params["enc_sa"]
    h, q1, k1, v1 = _stage_b2(q0, s0, feat_ico, kk0, vv0, pe["Wo"],
                              params["Wce1"], params["bce1"].reshape(1, HID),
                              p1["Wq"], p1["Wk"], p1["Wv"])

    sa_flat = sa_p.reshape(-1)
    kk = _take(k1, sa_flat).reshape(NP1, -1, HID)
    vv = _take(v1, sa_flat).reshape(NP1, -1, HID)
    h2, q2, k2, v2 = _stage_sa(h, q1, kk, vv, p1["Wo"],
                               p2["Wq"], p2["Wk"], p2["Wv"])

    d1, d2 = params["dec_sa"]
    kk = _take(k2, sa_flat).reshape(NP1, -1, HID)
    vv = _take(v2, sa_flat).reshape(NP1, -1, HID)
    g, q3, k3, v3 = _stage_b4(h2, q2, kk, vv, p2["Wo"],
                              params["Wlat"], params["blat"].reshape(1, LAT),
                              params["Wdlat"], params["bdlat"].reshape(1, HID),
                              feat_ico, params["Wcd1"],
                              params["bcd1"].reshape(1, HID),
                              d1["Wq"], d1["Wk"], d1["Wv"])

    kk = _take(k3, sa_flat).reshape(NP1, -1, HID)
    vv = _take(v3, sa_flat).reshape(NP1, -1, HID)
    g2, q4, k4, v4 = _stage_sa(g, q3, kk, vv, d1["Wo"],
                               d2["Wq"], d2["Wk"], d2["Wv"])

    kk = _take(k4, sa_flat).reshape(NP1, -1, HID)
    vv = _take(v4, sa_flat).reshape(NP1, -1, HID)
    Qd, Kd, Vd, Sd = _stage_b6(g2, q4, kk, vv, d2["Wo"],
                               pd["Wq"], pd["Wk"], pd["Wv"], pd["Wskip"])

    qd = _take(Qd, dec_q_idx)
    sd = _take(Sd, dec_q_idx)
    kkd = _take(Kd, dec_nbrs.reshape(-1)).reshape(N0, -1, HID)
    vvd = _take(Vd, dec_nbrs.reshape(-1)).reshape(N0, -1, HID)
    return _stage_c(sd, qd, kkd, vvd, pd["Wo"])


# speedup vs baseline: 2.3811x; 2.3811x over previous
import functools

import jax
import jax.numpy as jnp
import numpy as np
from jax import lax
from jax.experimental import pallas as pl
from jax.experimental.pallas import tpu as pltpu
from jax.experimental.pallas import tpu_sc as plsc

NW = 32

N0 = 50000
N1 = 10242
NP1 = 10496
CIN = 128
CC = 16
HID = 128
LAT = 32
HEADS = 8
DH = HID // HEADS
INV_SQRT_DH = 1.0 / float(np.sqrt(DH))
SAT_BOUND = 5.0

_MH = np.zeros((HID, HEADS), np.float32)
for _d in range(HID):
    _MH[_d, _d // DH] = 1.0
MH = jnp.asarray(_MH)
MHT = jnp.asarray(_MH.T.copy())


def _layernorm(x):
    m = x.mean(-1, keepdims=True)
    v = x.var(-1, keepdims=True)
    return (x - m) * jax.lax.rsqrt(v + 1e-5)


def _dot(a, b):
    return jnp.dot(a, b, preferred_element_type=jnp.float32)


def _attn(q, kk_ref, vv_ref, mh, mht, nk):
    den = None
    acc = None
    for k in range(nk):
        kkk = kk_ref[:, k, :]
        e = jnp.exp(_dot(q * kkk, mh))
        den = e if den is None else den + e
        contrib = _dot(e, mht) * vv_ref[:, k, :]
        acc = contrib if acc is None else acc + contrib
    return acc * _dot(1.0 / den, mht)


def _full(shape):
    return pl.BlockSpec(shape, lambda i: tuple(0 for _ in shape))


def _rows(block, width):
    return pl.BlockSpec((block, width), lambda i: (i, 0))


def _rows3(block, k, width):
    return pl.BlockSpec((block, k, width), lambda i: (i, 0, 0))


def _f32(shape):
    return jax.ShapeDtypeStruct(shape, jnp.float32)


def _stage_a_body(x_ref, feat_ref, wce_ref, bce_ref, wq_ref, wk_ref, wv_ref,
                  ws_ref, q_ref, k_ref, v_ref, s_ref):
    x0 = x_ref[...] + jnp.sin(_dot(feat_ref[...], wce_ref[...]) + bce_ref[...])
    ln = _layernorm(x0)
    q_ref[...] = _dot(ln, wq_ref[...]) * INV_SQRT_DH
    k_ref[...] = _dot(ln, wk_ref[...])
    v_ref[...] = _dot(ln, wv_ref[...])
    s_ref[...] = _dot(x0, ws_ref[...])


def _stage_a(x, feat, wce, bce, wq, wk, wv, ws, block=2000):
    n = x.shape[0]
    f = feat.shape[1]
    return pl.pallas_call(
        _stage_a_body,
        grid=(n // block,),
        in_specs=[_rows(block, CIN), _rows(block, f), _full((f, CIN)),
                  _full((1, CIN)), _full((CIN, HID)), _full((CIN, HID)),
                  _full((CIN, HID)), _full((CIN, HID))],
        out_specs=[_rows(block, HID)] * 4,
        out_shape=[_f32((n, HID))] * 4,
    )(x, feat, wce, bce, wq, wk, wv, ws)


def _stage_b2_body(q0_ref, s0_ref, feat_ref, kk_ref, vv_ref, mh_ref, mht_ref,
                   wo_ref, wce_ref, bce_ref, wq_ref, wk_ref, wv_ref,
                   h_ref, q_ref, k_ref, v_ref, *, nk):
    at = _attn(q0_ref[...], kk_ref, vv_ref, mh_ref[...], mht_ref[...], nk)
    h = s0_ref[...] + _dot(at, wo_ref[...]) \
        + jnp.sin(_dot(feat_ref[...], wce_ref[...]) + bce_ref[...])
    h_ref[...] = h
    hn = _layernorm(h)
    q_ref[...] = _dot(hn, wq_ref[...]) * INV_SQRT_DH
    k_ref[...] = _dot(hn, wk_ref[...])
    v_ref[...] = _dot(hn, wv_ref[...])


def _stage_b2(q0, s0, feat, kk, vv, wo, wce, bce, wq, wk, wv, block=328):
    n, nk, _ = kk.shape
    f = feat.shape[1]
    return pl.pallas_call(
        functools.partial(_stage_b2_body, nk=nk),
        grid=(n // block,),
        in_specs=[_rows(block, HID), _rows(block, HID), _rows(block, f),
                  _rows3(block, nk, HID), _rows3(block, nk, HID),
                  _full((HID, HEADS)), _full((HEADS, HID)),
                  _full((HID, HID)), _full((f, HID)), _full((1, HID)),
                  _full((HID, HID)), _full((HID, HID)), _full((HID, HID))],
        out_specs=[_rows(block, HID)] * 4,
        out_shape=[_f32((n, HID))] * 4,
    )(q0, s0, feat, kk, vv, MH, MHT, wo, wce, bce, wq, wk, wv)


def _stage_sa_body(h_ref, q_ref, kk_ref, vv_ref, mh_ref, mht_ref, wo_ref,
                   wq_ref, wk_ref, wv_ref,
                   h2_ref, q2_ref, k2_ref, v2_ref, *, nk):
    at = _attn(q_ref[...], kk_ref, vv_ref, mh_ref[...], mht_ref[...], nk)
    h2 = h_ref[...] + _dot(at, wo_ref[...])
    h2_ref[...] = h2
    hn = _layernorm(h2)
    q2_ref[...] = _dot(hn, wq_ref[...]) * INV_SQRT_DH
    k2_ref[...] = _dot(hn, wk_ref[...])
    v2_ref[...] = _dot(hn, wv_ref[...])


def _stage_sa(h, q, kk, vv, wo, wq, wk, wv, block=328):
    n, nk, _ = kk.shape
    return pl.pallas_call(
        functools.partial(_stage_sa_body, nk=nk),
        grid=(n // block,),
        in_specs=[_rows(block, HID), _rows(block, HID),
                  _rows3(block, nk, HID), _rows3(block, nk, HID),
                  _full((HID, HEADS)), _full((HEADS, HID)),
                  _full((HID, HID)), _full((HID, HID)), _full((HID, HID)),
                  _full((HID, HID))],
        out_specs=[_rows(block, HID)] * 4,
        out_shape=[_f32((n, HID))] * 4,
    )(h, q, kk, vv, MH, MHT, wo, wq, wk, wv)


def _stage_b4_body(h_ref, q_ref, kk_ref, vv_ref, mh_ref, mht_ref, wo_ref,
                   wlat_ref, blat_ref, wdlat_ref, bdlat_ref, feat_ref,
                   wcd_ref, bcd_ref, wq_ref, wk_ref, wv_ref,
                   g_ref, q2_ref, k2_ref, v2_ref, *, nk):
    at = _attn(q_ref[...], kk_ref, vv_ref, mh_ref[...], mht_ref[...], nk)
    h3 = h_ref[...] + _dot(at, wo_ref[...])
    z = _dot(h3, wlat_ref[...]) + blat_ref[...]
    z = z * jax.lax.rsqrt(1.0 + (z / SAT_BOUND) ** 2)
    g = _dot(z, wdlat_ref[...]) + bdlat_ref[...]
    g = g + jnp.sin(_dot(feat_ref[...], wcd_ref[...]) + bcd_ref[...])
    g_ref[...] = g
    gn = _layernorm(g)
    q2_ref[...] = _dot(gn, wq_ref[...]) * INV_SQRT_DH
    k2_ref[...] = _dot(gn, wk_ref[...])
    v2_ref[...] = _dot(gn, wv_ref[...])


def _stage_b4(h, q, kk, vv, wo, wlat, blat, wdlat, bdlat, feat, wcd, bcd,
              wq, wk, wv, block=328):
    n, nk, _ = kk.shape
    f = feat.shape[1]
    return pl.pallas_call(
        functools.partial(_stage_b4_body, nk=nk),
        grid=(n // block,),
        in_specs=[_rows(block, HID), _rows(block, HID),
                  _rows3(block, nk, HID), _rows3(block, nk, HID),
                  _full((HID, HEADS)), _full((HEADS, HID)),
                  _full((HID, HID)),
                  _full((HID, LAT)), _full((1, LAT)),
                  _full((LAT, HID)), _full((1, HID)),
                  _rows(block, f), _full((f, HID)), _full((1, HID)),
                  _full((HID, HID)), _full((HID, HID)), _full((HID, HID))],
        out_specs=[_rows(block, HID)] * 4,
        out_shape=[_f32((n, HID))] * 4,
    )(h, q, kk, vv, MH, MHT, wo, wlat, blat, wdlat, bdlat, feat, wcd, bcd,
      wq, wk, wv)


def _stage_b6_body(h_ref, q_ref, kk_ref, vv_ref, mh_ref, mht_ref, wo_ref,
                   wq_ref, wk_ref, wv_ref, ws_ref,
                   q2_ref, k2_ref, v2_ref, s2_ref, *, nk):
    at = _attn(q_ref[...], kk_ref, vv_ref, mh_ref[...], mht_ref[...], nk)
    g3 = h_ref[...] + _dot(at, wo_ref[...])
    gn = _layernorm(g3)
    q2_ref[...] = _dot(gn, wq_ref[...]) * INV_SQRT_DH
    k2_ref[...] = _dot(gn, wk_ref[...])
    v2_ref[...] = _dot(gn, wv_ref[...])
    s2_ref[...] = _dot(g3, ws_ref[...])


def _stage_b6(h, q, kk, vv, wo, wq, wk, wv, ws, block=328):
    n, nk, _ = kk.shape
    return pl.pallas_call(
        functools.partial(_stage_b6_body, nk=nk),
        grid=(n // block,),
        in_specs=[_rows(block, HID), _rows(block, HID),
                  _rows3(block, nk, HID), _rows3(block, nk, HID),
                  _full((HID, HEADS)), _full((HEADS, HID)),
                  _full((HID, HID)), _full((HID, HID)), _full((HID, HID)),
                  _full((HID, HID)), _full((HID, HID))],
        out_specs=[_rows(block, HID)] * 4,
        out_shape=[_f32((n, HID))] * 4,
    )(h, q, kk, vv, MH, MHT, wo, wq, wk, wv, ws)


def _stage_c_body(s_ref, q_ref, kk_ref, vv_ref, mh_ref, mht_ref, wo_ref,
                  out_ref, *, nk):
    at = _attn(q_ref[...], kk_ref, vv_ref, mh_ref[...], mht_ref[...], nk)
    out_ref[...] = s_ref[...] + _dot(at, wo_ref[...])


def _stage_c(s, q, kk, vv, wo, block=1000):
    n, nk, _ = kk.shape
    return pl.pallas_call(
        functools.partial(_stage_c_body, nk=nk),
        grid=(n // block,),
        in_specs=[_rows(block, HID), _rows(block, HID),
                  _rows3(block, nk, HID), _rows3(block, nk, HID),
                  _full((HID, HEADS)), _full((HEADS, HID)),
                  _full((HID, HID))],
        out_specs=_rows(block, CIN),
        out_shape=_f32((n, CIN)),
    )(s, q, kk, vv, MH, MHT, wo)


def _sc_gather_call(idx, tables, chunk):
    n_idx = idx.shape[0]
    per_w = n_idx // NW
    iters = per_w // chunk
    assert per_w * NW == n_idx and iters * chunk == per_w and chunk % 8 == 0
    nt = len(tables)
    mesh = plsc.VectorSubcoreMesh(core_axis_name="c", subcore_axis_name="s")
    scratch = ([pltpu.VMEM((chunk,), jnp.int32)]
               + [pltpu.VMEM((chunk, t.shape[1]), t.dtype) for t in tables]
               + [pltpu.SemaphoreType.DMA] * nt)
    out_type = [jax.ShapeDtypeStruct((n_idx, t.shape[1]), t.dtype)
                for t in tables]

    @functools.partial(pl.kernel, mesh=mesh, out_type=out_type,
                       scratch_types=scratch)
    def gather_k(*refs):
        tbl = refs[:nt]
        idx_hbm = refs[nt]
        outs = refs[nt + 1:2 * nt + 1]
        idxv = refs[2 * nt + 1]
        bufs = refs[2 * nt + 2:3 * nt + 2]
        sems = refs[3 * nt + 2:]
        wid = lax.axis_index("s") * 2 + lax.axis_index("c")
        wbase = wid * per_w

        def body(i, carry):
            base = wbase + i * chunk
            pltpu.sync_copy(idx_hbm.at[pl.ds(base, chunk)], idxv)
            cps = [pltpu.async_copy(tbl[t].at[idxv], bufs[t], sems[t])
                   for t in range(nt)]
            for c in cps:
                c.wait()
            for t in range(nt):
                pltpu.sync_copy(bufs[t], outs[t].at[pl.ds(base, chunk)])
            return carry

        lax.fori_loop(0, iters, body, 0)

    outs = gather_k(*tables, idx)
    return outs if nt > 1 else [outs]


NP0 = 50176


def kernel(x, context, coord4_grid, coord4_ico, params, pool_nbrs, sa_nbrs,
           dec_nbrs, enc_q_idx, dec_q_idx):
    pe = params["enc_pool"]
    pd = params["dec_pool"]

    feat_grid = jnp.concatenate([coord4_grid, context], axis=-1)
    Q0, K0, V0, S0 = _stage_a(x, feat_grid, params["Wce0"],
                              params["bce0"].reshape(1, CIN),
                              pe["Wq"], pe["Wk"], pe["Wv"], pe["Wskip"])

    pad1 = NP1 - N1
    enc_q_p = jnp.pad(enc_q_idx, (0, pad1)).astype(jnp.int32)
    sa_flat = jnp.pad(sa_nbrs, ((0, pad1), (0, 0))).astype(jnp.int32).reshape(-1)
    pool_flat = jnp.pad(pool_nbrs, ((0, pad1), (0, 0))).astype(jnp.int32).reshape(-1)

    context_p = jnp.pad(context, ((0, 0), (0, HID - CC)))
    q0, s0, ctx_ico = _sc_gather_call(enc_q_p, [Q0, S0, context_p], chunk=328)
    ctx_ico = ctx_ico[:, :CC]
    kk0, vv0 = _sc_gather_call(pool_flat, [K0, V0], chunk=328)
    kk0 = kk0.reshape(NP1, -1, HID)
    vv0 = vv0.reshape(NP1, -1, HID)

    feat_ico = jnp.concatenate(
        [jnp.pad(coord4_ico, ((0, pad1), (0, 0))), ctx_ico], axis=-1)

    p1, p2 = params["enc_sa"]
    h, q1, k1, v1 = _stage_b2(q0, s0, feat_ico, kk0, vv0, pe["Wo"],
                              params["Wce1"], params["bce1"].reshape(1, HID),
                              p1["Wq"], p1["Wk"], p1["Wv"])

    kk, vv = _sc_gather_call(sa_flat, [k1, v1], chunk=328)
    h2, q2, k2, v2 = _stage_sa(h, q1, kk.reshape(NP1, -1, HID),
                               vv.reshape(NP1, -1, HID), p1["Wo"],
                               p2["Wq"], p2["Wk"], p2["Wv"])

    d1, d2 = params["dec_sa"]
    kk, vv = _sc_gather_call(sa_flat, [k2, v2], chunk=328)
    g, q3, k3, v3 = _stage_b4(h2, q2, kk.reshape(NP1, -1, HID),
                              vv.reshape(NP1, -1, HID), p2["Wo"],
                              params["Wlat"], params["blat"].reshape(1, LAT),
                              params["Wdlat"], params["bdlat"].reshape(1, HID),
                              feat_ico, params["Wcd1"],
                              params["bcd1"].reshape(1, HID),
                              d1["Wq"], d1["Wk"], d1["Wv"])

    kk, vv = _sc_gather_call(sa_flat, [k3, v3], chunk=328)
    g2, q4, k4, v4 = _stage_sa(g, q3, kk.reshape(NP1, -1, HID),
                               vv.reshape(NP1, -1, HID), d1["Wo"],
                               d2["Wq"], d2["Wk"], d2["Wv"])

    kk, vv = _sc_gather_call(sa_flat, [k4, v4], chunk=328)
    Qd, Kd, Vd, Sd = _stage_b6(g2, q4, kk.reshape(NP1, -1, HID),
                               vv.reshape(NP1, -1, HID), d2["Wo"],
                               pd["Wq"], pd["Wk"], pd["Wv"], pd["Wskip"])

    pad0 = NP0 - N0
    dec_q_p = jnp.pad(dec_q_idx, (0, pad0)).astype(jnp.int32)
    dec_flat = jnp.pad(dec_nbrs, ((0, pad0), (0, 0))).astype(jnp.int32).reshape(-1)
    qd, sd = _sc_gather_call(dec_q_p, [Qd, Sd], chunk=392)
    kkd, vvd = _sc_gather_call(dec_flat, [Kd, Vd], chunk=392)
    out = _stage_c(sd, qd, kkd.reshape(NP0, -1, HID),
                   vvd.reshape(NP0, -1, HID), pd["Wo"], block=784)
    return out[:N0]


# speedup vs baseline: 2.3922x; 1.0047x over previous
import functools

import jax
import jax.numpy as jnp
import numpy as np
from jax import lax
from jax.experimental import pallas as pl
from jax.experimental.pallas import tpu as pltpu
from jax.experimental.pallas import tpu_sc as plsc

NW = 32

N0 = 50000
N1 = 10242
NP1 = 10496
CIN = 128
CC = 16
HID = 128
LAT = 32
HEADS = 8
DH = HID // HEADS
INV_SQRT_DH = 1.0 / float(np.sqrt(DH))
SAT_BOUND = 5.0

_MH = np.zeros((HID, HEADS), np.float32)
for _d in range(HID):
    _MH[_d, _d // DH] = 1.0
MH = jnp.asarray(_MH)
MHT = jnp.asarray(_MH.T.copy())


def _layernorm(x):
    m = x.mean(-1, keepdims=True)
    v = x.var(-1, keepdims=True)
    return (x - m) * jax.lax.rsqrt(v + 1e-5)


def _dot(a, b):
    return jnp.dot(a, b, preferred_element_type=jnp.float32)


def _attn(q, kk_ref, vv_ref, mh, mht, nk):
    b = q.shape[0]
    kkf = kk_ref[...].reshape(b * nk, HID)
    vvf = vv_ref[...].reshape(b * nk, HID)
    qrep = jnp.broadcast_to(q[:, None, :], (b, nk, HID)).reshape(b * nk, HID)
    lexp = jnp.exp(_dot(qrep * kkf, mh))
    wfull = _dot(lexp, mht)
    w3 = wfull.reshape(b, nk, HID)
    wv3 = (wfull * vvf).reshape(b, nk, HID)
    num = wv3[:, 0, :]
    den = w3[:, 0, :]
    for k in range(1, nk):
        num = num + wv3[:, k, :]
        den = den + w3[:, k, :]
    return num / den


def _full(shape):
    return pl.BlockSpec(shape, lambda i: tuple(0 for _ in shape))


def _rows(block, width):
    return pl.BlockSpec((block, width), lambda i: (i, 0))


def _rows3(block, k, width):
    return pl.BlockSpec((block, k, width), lambda i: (i, 0, 0))


def _f32(shape):
    return jax.ShapeDtypeStruct(shape, jnp.float32)


def _stage_a_body(x_ref, feat_ref, wce_ref, bce_ref, wq_ref, wk_ref, wv_ref,
                  ws_ref, q_ref, k_ref, v_ref, s_ref):
    x0 = x_ref[...] + jnp.sin(_dot(feat_ref[...], wce_ref[...]) + bce_ref[...])
    ln = _layernorm(x0)
    q_ref[...] = _dot(ln, wq_ref[...]) * INV_SQRT_DH
    k_ref[...] = _dot(ln, wk_ref[...])
    v_ref[...] = _dot(ln, wv_ref[...])
    s_ref[...] = _dot(x0, ws_ref[...])


def _stage_a(x, feat, wce, bce, wq, wk, wv, ws, block=5000):
    n = x.shape[0]
    f = feat.shape[1]
    return pl.pallas_call(
        _stage_a_body,
        grid=(n // block,),
        in_specs=[_rows(block, CIN), _rows(block, f), _full((f, CIN)),
                  _full((1, CIN)), _full((CIN, HID)), _full((CIN, HID)),
                  _full((CIN, HID)), _full((CIN, HID))],
        out_specs=[_rows(block, HID)] * 4,
        out_shape=[_f32((n, HID))] * 4,
    )(x, feat, wce, bce, wq, wk, wv, ws)


def _stage_b2_body(q0_ref, s0_ref, feat_ref, kk_ref, vv_ref, mh_ref, mht_ref,
                   wo_ref, wce_ref, bce_ref, wq_ref, wk_ref, wv_ref,
                   h_ref, q_ref, k_ref, v_ref, *, nk):
    at = _attn(q0_ref[...], kk_ref, vv_ref, mh_ref[...], mht_ref[...], nk)
    h = s0_ref[...] + _dot(at, wo_ref[...]) \
        + jnp.sin(_dot(feat_ref[...], wce_ref[...]) + bce_ref[...])
    h_ref[...] = h
    hn = _layernorm(h)
    q_ref[...] = _dot(hn, wq_ref[...]) * INV_SQRT_DH
    k_ref[...] = _dot(hn, wk_ref[...])
    v_ref[...] = _dot(hn, wv_ref[...])


def _stage_b2(q0, s0, feat, kk, vv, wo, wce, bce, wq, wk, wv, block=328):
    n, nk, _ = kk.shape
    f = feat.shape[1]
    return pl.pallas_call(
        functools.partial(_stage_b2_body, nk=nk),
        grid=(n // block,),
        in_specs=[_rows(block, HID), _rows(block, HID), _rows(block, f),
                  _rows3(block, nk, HID), _rows3(block, nk, HID),
                  _full((HID, HEADS)), _full((HEADS, HID)),
                  _full((HID, HID)), _full((f, HID)), _full((1, HID)),
                  _full((HID, HID)), _full((HID, HID)), _full((HID, HID))],
        out_specs=[_rows(block, HID)] * 4,
        out_shape=[_f32((n, HID))] * 4,
    )(q0, s0, feat, kk, vv, MH, MHT, wo, wce, bce, wq, wk, wv)


def _stage_sa_body(h_ref, q_ref, kk_ref, vv_ref, mh_ref, mht_ref, wo_ref,
                   wq_ref, wk_ref, wv_ref,
                   h2_ref, q2_ref, k2_ref, v2_ref, *, nk):
    at = _attn(q_ref[...], kk_ref, vv_ref, mh_ref[...], mht_ref[...], nk)
    h2 = h_ref[...] + _dot(at, wo_ref[...])
    h2_ref[...] = h2
    hn = _layernorm(h2)
    q2_ref[...] = _dot(hn, wq_ref[...]) * INV_SQRT_DH
    k2_ref[...] = _dot(hn, wk_ref[...])
    v2_ref[...] = _dot(hn, wv_ref[...])


def _stage_sa(h, q, kk, vv, wo, wq, wk, wv, block=656):
    n, nk, _ = kk.shape
    return pl.pallas_call(
        functools.partial(_stage_sa_body, nk=nk),
        grid=(n // block,),
        in_specs=[_rows(block, HID), _rows(block, HID),
                  _rows3(block, nk, HID), _rows3(block, nk, HID),
                  _full((HID, HEADS)), _full((HEADS, HID)),
                  _full((HID, HID)), _full((HID, HID)), _full((HID, HID)),
                  _full((HID, HID))],
        out_specs=[_rows(block, HID)] * 4,
        out_shape=[_f32((n, HID))] * 4,
    )(h, q, kk, vv, MH, MHT, wo, wq, wk, wv)


def _stage_b4_body(h_ref, q_ref, kk_ref, vv_ref, mh_ref, mht_ref, wo_ref,
                   wlat_ref, blat_ref, wdlat_ref, bdlat_ref, feat_ref,
                   wcd_ref, bcd_ref, wq_ref, wk_ref, wv_ref,
                   g_ref, q2_ref, k2_ref, v2_ref, *, nk):
    at = _attn(q_ref[...], kk_ref, vv_ref, mh_ref[...], mht_ref[...], nk)
    h3 = h_ref[...] + _dot(at, wo_ref[...])
    z = _dot(h3, wlat_ref[...]) + blat_ref[...]
    z = z * jax.lax.rsqrt(1.0 + (z / SAT_BOUND) ** 2)
    g = _dot(z, wdlat_ref[...]) + bdlat_ref[...]
    g = g + jnp.sin(_dot(feat_ref[...], wcd_ref[...]) + bcd_ref[...])
    g_ref[...] = g
    gn = _layernorm(g)
    q2_ref[...] = _dot(gn, wq_ref[...]) * INV_SQRT_DH
    k2_ref[...] = _dot(gn, wk_ref[...])
    v2_ref[...] = _dot(gn, wv_ref[...])


def _stage_b4(h, q, kk, vv, wo, wlat, blat, wdlat, bdlat, feat, wcd, bcd,
              wq, wk, wv, block=656):
    n, nk, _ = kk.shape
    f = feat.shape[1]
    return pl.pallas_call(
        functools.partial(_stage_b4_body, nk=nk),
        grid=(n // block,),
        in_specs=[_rows(block, HID), _rows(block, HID),
                  _rows3(block, nk, HID), _rows3(block, nk, HID),
                  _full((HID, HEADS)), _full((HEADS, HID)),
                  _full((HID, HID)),
                  _full((HID, LAT)), _full((1, LAT)),
                  _full((LAT, HID)), _full((1, HID)),
                  _rows(block, f), _full((f, HID)), _full((1, HID)),
                  _full((HID, HID)), _full((HID, HID)), _full((HID, HID))],
        out_specs=[_rows(block, HID)] * 4,
        out_shape=[_f32((n, HID))] * 4,
    )(h, q, kk, vv, MH, MHT, wo, wlat, blat, wdlat, bdlat, feat, wcd, bcd,
      wq, wk, wv)


def _stage_b6_body(h_ref, q_ref, kk_ref, vv_ref, mh_ref, mht_ref, wo_ref,
                   wq_ref, wk_ref, wv_ref, ws_ref,
                   q2_ref, k2_ref, v2_ref, s2_ref, *, nk):
    at = _attn(q_ref[...], kk_ref, vv_ref, mh_ref[...], mht_ref[...], nk)
    g3 = h_ref[...] + _dot(at, wo_ref[...])
    gn = _layernorm(g3)
    q2_ref[...] = _dot(gn, wq_ref[...]) * INV_SQRT_DH
    k2_ref[...] = _dot(gn, wk_ref[...])
    v2_ref[...] = _dot(gn, wv_ref[...])
    s2_ref[...] = _dot(g3, ws_ref[...])


def _stage_b6(h, q, kk, vv, wo, wq, wk, wv, ws, block=656):
    n, nk, _ = kk.shape
    return pl.pallas_call(
        functools.partial(_stage_b6_body, nk=nk),
        grid=(n // block,),
        in_specs=[_rows(block, HID), _rows(block, HID),
                  _rows3(block, nk, HID), _rows3(block, nk, HID),
                  _full((HID, HEADS)), _full((HEADS, HID)),
                  _full((HID, HID)), _full((HID, HID)), _full((HID, HID)),
                  _full((HID, HID)), _full((HID, HID))],
        out_specs=[_rows(block, HID)] * 4,
        out_shape=[_f32((n, HID))] * 4,
    )(h, q, kk, vv, MH, MHT, wo, wq, wk, wv, ws)


def _stage_c_body(s_ref, q_ref, kk_ref, vv_ref, mh_ref, mht_ref, wo_ref,
                  out_ref, *, nk):
    at = _attn(q_ref[...], kk_ref, vv_ref, mh_ref[...], mht_ref[...], nk)
    out_ref[...] = s_ref[...] + _dot(at, wo_ref[...])


def _stage_c(s, q, kk, vv, wo, block=1568):
    n, nk, _ = kk.shape
    return pl.pallas_call(
        functools.partial(_stage_c_body, nk=nk),
        grid=(n // block,),
        in_specs=[_rows(block, HID), _rows(block, HID),
                  _rows3(block, nk, HID), _rows3(block, nk, HID),
                  _full((HID, HEADS)), _full((HEADS, HID)),
                  _full((HID, HID))],
        out_specs=_rows(block, CIN),
        out_shape=_f32((n, CIN)),
    )(s, q, kk, vv, MH, MHT, wo)


def _sc_gather_call(idx, tables, chunk):
    n_idx = idx.shape[0]
    per_w = n_idx // NW
    iters = per_w // chunk
    assert per_w * NW == n_idx and iters * chunk == per_w and chunk % 8 == 0
    nt = len(tables)
    mesh = plsc.VectorSubcoreMesh(core_axis_name="c", subcore_axis_name="s")
    scratch = ([pltpu.VMEM((chunk,), jnp.int32)]
               + [pltpu.VMEM((chunk, t.shape[1]), t.dtype) for t in tables]
               + [pltpu.SemaphoreType.DMA] * nt)
    out_type = [jax.ShapeDtypeStruct((n_idx, t.shape[1]), t.dtype)
                for t in tables]

    @functools.partial(pl.kernel, mesh=mesh, out_type=out_type,
                       scratch_types=scratch)
    def gather_k(*refs):
        tbl = refs[:nt]
        idx_hbm = refs[nt]
        outs = refs[nt + 1:2 * nt + 1]
        idxv = refs[2 * nt + 1]
        bufs = refs[2 * nt + 2:3 * nt + 2]
        sems = refs[3 * nt + 2:]
        wid = lax.axis_index("s") * 2 + lax.axis_index("c")
        wbase = wid * per_w

        def body(i, carry):
            base = wbase + i * chunk
            pltpu.sync_copy(idx_hbm.at[pl.ds(base, chunk)], idxv)
            cps = [pltpu.async_copy(tbl[t].at[idxv], bufs[t], sems[t])
                   for t in range(nt)]
            for c in cps:
                c.wait()
            for t in range(nt):
                pltpu.sync_copy(bufs[t], outs[t].at[pl.ds(base, chunk)])
            return carry

        lax.fori_loop(0, iters, body, 0)

    outs = gather_k(*tables, idx)
    return outs if nt > 1 else [outs]


NP0 = 50176


def kernel(x, context, coord4_grid, coord4_ico, params, pool_nbrs, sa_nbrs,
           dec_nbrs, enc_q_idx, dec_q_idx):
    pe = params["enc_pool"]
    pd = params["dec_pool"]

    feat_grid = jnp.concatenate([coord4_grid, context], axis=-1)
    Q0, K0, V0, S0 = _stage_a(x, feat_grid, params["Wce0"],
                              params["bce0"].reshape(1, CIN),
                              pe["Wq"], pe["Wk"], pe["Wv"], pe["Wskip"])

    pad1 = NP1 - N1
    enc_q_p = jnp.pad(enc_q_idx, (0, pad1)).astype(jnp.int32)
    sa_flat = jnp.pad(sa_nbrs, ((0, pad1), (0, 0))).astype(jnp.int32).reshape(-1)
    pool_flat = jnp.pad(pool_nbrs, ((0, pad1), (0, 0))).astype(jnp.int32).reshape(-1)

    context_p = jnp.pad(context, ((0, 0), (0, HID - CC)))
    q0, s0, ctx_ico = _sc_gather_call(enc_q_p, [Q0, S0, context_p], chunk=328)
    ctx_ico = ctx_ico[:, :CC]
    kk0, vv0 = _sc_gather_call(pool_flat, [K0, V0], chunk=328)
    kk0 = kk0.reshape(NP1, -1, HID)
    vv0 = vv0.reshape(NP1, -1, HID)

    feat_ico = jnp.concatenate(
        [jnp.pad(coord4_ico, ((0, pad1), (0, 0))), ctx_ico], axis=-1)

    p1, p2 = params["enc_sa"]
    h, q1, k1, v1 = _stage_b2(q0, s0, feat_ico, kk0, vv0, pe["Wo"],
                              params["Wce1"], params["bce1"].reshape(1, HID),
                              p1["Wq"], p1["Wk"], p1["Wv"])

    kk, vv = _sc_gather_call(sa_flat, [k1, v1], chunk=328)
    h2, q2, k2, v2 = _stage_sa(h, q1, kk.reshape(NP1, -1, HID),
                               vv.reshape(NP1, -1, HID), p1["Wo"],
                               p2["Wq"], p2["Wk"], p2["Wv"])

    d1, d2 = params["dec_sa"]
    kk, vv = _sc_gather_call(sa_flat, [k2, v2], chunk=328)
    g, q3, k3, v3 = _stage_b4(h2, q2, kk.reshape(NP1, -1, HID),
                              vv.reshape(NP1, -1, HID), p2["Wo"],
                              params["Wlat"], params["blat"].reshape(1, LAT),
                              params["Wdlat"], params["bdlat"].reshape(1, HID),
                              feat_ico, params["Wcd1"],
                              params["bcd1"].reshape(1, HID),
                              d1["Wq"], d1["Wk"], d1["Wv"])

    kk, vv = _sc_gather_call(sa_flat, [k3, v3], chunk=328)
    g2, q4, k4, v4 = _stage_sa(g, q3, kk.reshape(NP1, -1, HID),
                               vv.reshape(NP1, -1, HID), d1["Wo"],
                               d2["Wq"], d2["Wk"], d2["Wv"])

    kk, vv = _sc_gather_call(sa_flat, [k4, v4], chunk=328)
    Qd, Kd, Vd, Sd = _stage_b6(g2, q4, kk.reshape(NP1, -1, HID),
                               vv.reshape(NP1, -1, HID), d2["Wo"],
                               pd["Wq"], pd["Wk"], pd["Wv"], pd["Wskip"])

    pad0 = NP0 - N0
    dec_q_p = jnp.pad(dec_q_idx, (0, pad0)).astype(jnp.int32)
    dec_flat = jnp.pad(dec_nbrs, ((0, pad0), (0, 0))).astype(jnp.int32).reshape(-1)
    qd, sd = _sc_gather_call(dec_q_p, [Qd, Sd], chunk=392)
    kkd, vvd = _sc_gather_call(dec_flat, [Kd, Vd], chunk=392)
    out = _stage_c(sd, qd, kkd.reshape(NP0, -1, HID),
                   vvd.reshape(NP0, -1, HID), pd["Wo"], block=784)
    return out[:N0]
